# baseline (speedup 1.0000x reference)
import functools

import jax
import jax.numpy as jnp
from jax import lax
from jax.experimental import pallas as pl
from jax.experimental.pallas import tpu as pltpu
from jax.experimental.pallas import tpu_sc as plsc

BN = 32
Q = 64
P = 64
R = 64
LANES = 16
NCHUNK = R // LANES
TOTAL = BN * Q * P * R


U = 2
NSUP = Q // U


@functools.partial(
    pl.kernel,
    out_type=jax.ShapeDtypeStruct((BN, LANES), jnp.float32),
    mesh=plsc.VectorSubcoreMesh(core_axis_name="c", subcore_axis_name="s"),
    compiler_params=pltpu.CompilerParams(needs_layout_passes=False),
    scratch_types=[
        pltpu.VMEM((U * P * R,), jnp.float32),
        pltpu.VMEM((U * P * R,), jnp.float32),
        pltpu.VMEM((P, U, R), jnp.int32),
        pltpu.VMEM((P, U, R), jnp.int32),
        pltpu.VMEM((P, U, R), jnp.float32),
        pltpu.VMEM((P, U, R), jnp.float32),
        pltpu.VMEM((LANES,), jnp.float32),
        pltpu.SemaphoreType.DMA,
        pltpu.SemaphoreType.DMA,
    ],
)
def _pjc_sc(x_hbm, tgt_hbm, idx_hbm, out_hbm,
            xq0, xq1, idx0, idx1, tgt0, tgt1, acc_v, sem0, sem1):
    w = lax.axis_index("s") * 2 + lax.axis_index("c")

    def issue(s, xq, idxb, tgtb, sem):
        u = s * U
        pltpu.async_copy(x_hbm.at[pl.ds((w * Q + u) * P * R, U * P * R)], xq, sem)
        pltpu.async_copy(idx_hbm.at[w, :, pl.ds(u, U), :], idxb, sem)
        pltpu.async_copy(tgt_hbm.at[w, :, pl.ds(u, U), :], tgtb, sem)

    def drain(s, xq, idxb, tgtb, sem):
        u = s * U
        pltpu.make_async_copy(
            x_hbm.at[pl.ds((w * Q + u) * P * R, U * P * R)], xq, sem).wait()
        pltpu.make_async_copy(idx_hbm.at[w, :, pl.ds(u, U), :], idxb, sem).wait()
        pltpu.make_async_copy(tgt_hbm.at[w, :, pl.ds(u, U), :], tgtb, sem).wait()

    def unit_compute(xq, idxb, tgtb, accs):
        def body(p, a):
            a = list(a)
            for j in range(U):
                bias = p + j * (P * R)
                for c in range(NCHUNK):
                    iv = idxb[p, j, pl.ds(c * LANES, LANES)]
                    fl = iv * R + bias
                    g = plsc.load_gather(xq, [fl])
                    t = tgtb[p, j, pl.ds(c * LANES, LANES)]
                    d = g - t
                    a[c] = a[c] + d * d
            return tuple(a)
        return plsc.parallel_loop(0, P, unroll=2, carry=accs)(body)

    issue(0, xq0, idx0, tgt0, sem0)
    issue(1, xq1, idx1, tgt1, sem1)

    def pair_body(i, accs):
        s0 = 2 * i
        drain(s0, xq0, idx0, tgt0, sem0)
        accs = unit_compute(xq0, idx0, tgt0, accs)

        @pl.when(s0 + 2 < NSUP)
        def _():
            issue(s0 + 2, xq0, idx0, tgt0, sem0)

        s1 = s0 + 1
        drain(s1, xq1, idx1, tgt1, sem1)
        accs = unit_compute(xq1, idx1, tgt1, accs)

        @pl.when(s1 + 2 < NSUP)
        def _():
            issue(s1 + 2, xq1, idx1, tgt1, sem1)

        return accs

    zero = jnp.zeros((LANES,), jnp.float32)
    accs = lax.fori_loop(0, NSUP // 2, pair_body, (zero, zero, zero, zero))
    total = (accs[0] + accs[1]) + (accs[2] + accs[3])
    acc_v[...] = total * (1.0 / TOTAL)
    pltpu.sync_copy(acc_v, out_hbm.at[w])


def kernel(input, target, idx_expanded):
    x = input.reshape(BN * Q * P * R)
    tgt = target.reshape(BN, P, Q, R)
    idx = idx_expanded.reshape(BN, P, Q, R)
    partial = _pjc_sc(x, tgt, idx)
    return jnp.sum(partial)


# speedup vs baseline: 1.0356x; 1.0356x over previous
import functools

import jax
import jax.numpy as jnp
from jax import lax
from jax.experimental import pallas as pl
from jax.experimental.pallas import tpu as pltpu
from jax.experimental.pallas import tpu_sc as plsc

BN = 32
Q = 64
P = 64
R = 64
LANES = 16
NCHUNK = R // LANES
TOTAL = BN * Q * P * R
NBUF = 4


@functools.partial(
    pl.kernel,
    out_type=jax.ShapeDtypeStruct((BN, LANES), jnp.float32),
    mesh=plsc.VectorSubcoreMesh(core_axis_name="c", subcore_axis_name="s"),
    compiler_params=pltpu.CompilerParams(needs_layout_passes=False),
    scratch_types=(
        [pltpu.VMEM((P * R,), jnp.float32) for _ in range(NBUF)]
        + [pltpu.VMEM((P, R), jnp.int32) for _ in range(NBUF)]
        + [pltpu.VMEM((P, R), jnp.float32) for _ in range(NBUF)]
        + [pltpu.VMEM((LANES,), jnp.float32)]
        + [pltpu.SemaphoreType.DMA for _ in range(NBUF)]
    ),
)
def _pjc_sc(x_hbm, tgt_hbm, idx_hbm, out_hbm, *refs):
    xqs = refs[0:NBUF]
    idxs = refs[NBUF:2 * NBUF]
    tgts = refs[2 * NBUF:3 * NBUF]
    acc_v = refs[3 * NBUF]
    sems = refs[3 * NBUF + 1:3 * NBUF + 1 + NBUF]

    w = lax.axis_index("s") * 2 + lax.axis_index("c")

    def issue(u, b):
        pltpu.async_copy(x_hbm.at[pl.ds((w * Q + u) * P * R, P * R)], xqs[b], sems[b])
        pltpu.async_copy(idx_hbm.at[w, :, u, :], idxs[b], sems[b])
        pltpu.async_copy(tgt_hbm.at[w, :, u, :], tgts[b], sems[b])

    def drain(u, b):
        pltpu.make_async_copy(
            x_hbm.at[pl.ds((w * Q + u) * P * R, P * R)], xqs[b], sems[b]).wait()
        pltpu.make_async_copy(idx_hbm.at[w, :, u, :], idxs[b], sems[b]).wait()
        pltpu.make_async_copy(tgt_hbm.at[w, :, u, :], tgts[b], sems[b]).wait()

    def unit_compute(b, accs):
        xq, idxb, tgtb = xqs[b], idxs[b], tgts[b]

        def body(p, a):
            a = list(a)
            for c in range(NCHUNK):
                iv = idxb[p, pl.ds(c * LANES, LANES)]
                fl = iv * R + p
                g = plsc.load_gather(xq, [fl])
                t = tgtb[p, pl.ds(c * LANES, LANES)]
                d = g - t
                a[c] = a[c] + d * d
            return tuple(a)
        return plsc.parallel_loop(0, P, unroll=4, carry=accs)(body)

    for b in range(NBUF):
        issue(b, b)

    def group_body(g, accs):
        u0 = g * NBUF
        for b in range(NBUF):
            u = u0 + b
            drain(u, b)
            accs = unit_compute(b, accs)

            @pl.when(u + NBUF < Q)
            def _():
                issue(u + NBUF, b)
        return accs

    zero = jnp.zeros((LANES,), jnp.float32)
    accs = lax.fori_loop(0, Q // NBUF, group_body, (zero, zero, zero, zero))
    total = (accs[0] + accs[1]) + (accs[2] + accs[3])
    acc_v[...] = total * (1.0 / TOTAL)
    pltpu.sync_copy(acc_v, out_hbm.at[w])


def kernel(input, target, idx_expanded):
    x = input.reshape(BN * Q * P * R)
    tgt = target.reshape(BN, P, Q, R)
    idx = idx_expanded.reshape(BN, P, Q, R)
    partial = _pjc_sc(x, tgt, idx)
    return jnp.sum(partial)
